# baseline (speedup 1.0000x reference)
import functools

import jax
import jax.numpy as jnp
from jax import lax
from jax.experimental import pallas as pl
from jax.experimental.pallas import tpu as pltpu
from jax.experimental.pallas import tpu_sc as plsc

N = 10000
NPAD = 10112
E = 320000
EPAD = 331776
D = 128
NEG = 0.2
NWORK = 32
CHUNKS = 216
B = 48
SCH = 8
NSUP = CHUNKS // SCH
PAIRS = CHUNKS // 2
PPS = SCH // 2


def _proj_body(x_ref, w_ref, b_ref, att_ref, feat_ref, epre_ref):
    feat = jnp.dot(x_ref[...], w_ref[...], preferred_element_type=jnp.float32)
    feat = feat + b_ref[...]
    feat_ref[pl.ds(0, N), :] = feat
    feat_ref[pl.ds(N, NPAD - N), :] = jnp.broadcast_to(b_ref[...],
                                                       (NPAD - N, D))
    epre = jnp.dot(feat, att_ref[...], preferred_element_type=jnp.float32)
    epre_ref[pl.ds(0, N), :] = epre
    epre_ref[pl.ds(N, NPAD - N), :] = jnp.zeros((NPAD - N, 2), jnp.float32)


def _project(x, W, b2, att):
    return pl.pallas_call(
        _proj_body,
        out_shape=(
            jax.ShapeDtypeStruct((NPAD, D), jnp.float32),
            jax.ShapeDtypeStruct((NPAD, 2), jnp.float32),
        ),
    )(x, W, b2, att)


def _edge_body(feat_hbm, esrc_hbm, etar_hbm, src_hbm, tar_hbm,
               out_ref, outs_ref,
               acc_s, esrc_v, etar_v, srcv, tarv, rows0, rows1, wbuf, s_v,
               gsem0, gsem1, ssem0, ssem1, isem):
    cid = lax.axis_index("c")
    sid = lax.axis_index("s")
    wid = cid * 16 + sid

    pltpu.sync_copy(esrc_hbm, esrc_v)
    pltpu.sync_copy(etar_hbm, etar_v)

    zero16 = jnp.zeros((16,), jnp.float32)
    lane = lax.broadcasted_iota(jnp.int32, (16,), 0)

    def _zs(j, _):
        s_v[pl.ds(j * 16, 16)] = zero16
        return 0

    lax.fori_loop(0, NPAD // 16, _zs, 0)

    def _zrow(j, _):
        for c in range(D // 16):
            rows0[j, pl.ds(c * 16, 16)] = zero16
        return 0

    lax.fori_loop(0, B, _zrow, 0)
    rows_per_tile = NPAD // 16
    base = sid * rows_per_tile
    for k in range(rows_per_tile // B):
        pltpu.sync_copy(rows0, acc_s.at[pl.ds(base + k * B, B)])
    rem = rows_per_tile % B
    if rem:
        pltpu.sync_copy(rows0.at[pl.ds(0, rem)],
                        acc_s.at[pl.ds(base + (rows_per_tile // B) * B, rem)])
    plsc.subcore_barrier()

    def _refill_issue(sup, half):
        pltpu.async_copy(src_hbm.at[wid, pl.ds(sup * SCH, SCH)],
                         srcv.at[half], isem)
        pltpu.async_copy(tar_hbm.at[wid, pl.ds(sup * SCH, SCH)],
                         tarv.at[half], isem)

    def _refill_wait(half):
        pltpu.make_async_copy(src_hbm.at[wid, pl.ds(0, SCH)],
                              srcv.at[half], isem).wait()
        pltpu.make_async_copy(tar_hbm.at[wid, pl.ds(0, SCH)],
                              tarv.at[half], isem).wait()

    def _weights(h, gg):
        for i in range(B // 16):
            s16 = srcv[h, gg, pl.ds(i * 16, 16)]
            t16 = tarv[h, gg, pl.ds(i * 16, 16)]
            z = (plsc.load_gather(esrc_v, [s16])
                 + plsc.load_gather(etar_v, [t16]))
            w16 = jnp.exp(jnp.maximum(z, NEG * z))
            wbuf[pl.ds(i * 16, 16)] = w16
            plsc.addupdate_scatter(s_v, [t16], w16)

    def _scale(rows):
        def _edge(j, _):
            wsp = plsc.load_gather(wbuf, [lane * 0 + j])
            for c in range(D // 16):
                rows[j, pl.ds(c * 16, 16)] = rows[j, pl.ds(c * 16, 16)] * wsp
            return 0

        lax.fori_loop(0, B, _edge, 0, unroll=2)

    pltpu.sync_copy(src_hbm.at[wid, pl.ds(0, SCH)], srcv.at[0])
    pltpu.sync_copy(tar_hbm.at[wid, pl.ds(0, SCH)], tarv.at[0])
    pltpu.async_copy(feat_hbm.at[srcv.at[0, 0]], rows0, gsem0)
    _refill_issue(1, 1)

    def _pair(p, _):
        sc = p // PPS
        pin = p % PPS
        h = sc & 1
        ga = (2 * p) % SCH
        gb = ga + 1

        @pl.when(jnp.logical_and(jnp.logical_and(pin == 0, p > 0),
                                 sc + 1 < NSUP))
        def _():
            _refill_issue(sc + 1, 1 - h)

        d_gb = pltpu.async_copy(feat_hbm.at[srcv.at[h, gb]], rows1, gsem1)

        _weights(h, ga)
        pltpu.make_async_copy(feat_hbm.at[srcv.at[h, ga]], rows0, gsem0).wait()
        _scale(rows0)
        d_sa = pltpu.async_copy(rows0, acc_s.at[tarv.at[h, ga]], ssem0,
                                add=True)

        _weights(h, gb)
        d_gb.wait()
        _scale(rows1)
        d_sb = pltpu.async_copy(rows1, acc_s.at[tarv.at[h, gb]], ssem1,
                                add=True)

        d_sa.wait()

        @pl.when(jnp.logical_and(pin == PPS - 1, p + 1 < PAIRS))
        def _():
            _refill_wait(1 - h)

        @pl.when(p + 1 < PAIRS)
        def _():
            nh = ((p + 1) // PPS) & 1
            nga = (2 * p + 2) % SCH
            pltpu.async_copy(feat_hbm.at[srcv.at[nh, nga]], rows0, gsem0)

        d_sb.wait()
        return 0

    lax.fori_loop(0, PAIRS, _pair, 0)
    plsc.subcore_barrier()

    for k in range(rows_per_tile // B):
        off = base + k * B
        pltpu.sync_copy(acc_s.at[pl.ds(off, B)], out_ref.at[cid, pl.ds(off, B)])
    if rem:
        off = base + (rows_per_tile // B) * B
        pltpu.sync_copy(acc_s.at[pl.ds(off, rem)],
                        out_ref.at[cid, pl.ds(off, rem)])
    pltpu.sync_copy(s_v, outs_ref.at[cid, sid])


def _edge_aggregate(feat, e_src, e_tar, src_p, tar_p):
    mesh = plsc.VectorSubcoreMesh(core_axis_name="c", subcore_axis_name="s")
    k = functools.partial(
        pl.kernel,
        out_type=(
            jax.ShapeDtypeStruct((2, NPAD, D), jnp.float32),
            jax.ShapeDtypeStruct((2, 16, NPAD), jnp.float32),
        ),
        mesh=mesh,
        compiler_params=pltpu.CompilerParams(needs_layout_passes=False),
        scratch_types=[
            pltpu.VMEM_SHARED((NPAD, D), jnp.float32),
            pltpu.VMEM((NPAD,), jnp.float32),
            pltpu.VMEM((NPAD,), jnp.float32),
            pltpu.VMEM((2, SCH, B), jnp.int32),
            pltpu.VMEM((2, SCH, B), jnp.int32),
            pltpu.VMEM((B, D), jnp.float32),
            pltpu.VMEM((B, D), jnp.float32),
            pltpu.VMEM((B,), jnp.float32),
            pltpu.VMEM((NPAD,), jnp.float32),
            pltpu.SemaphoreType.DMA,
            pltpu.SemaphoreType.DMA,
            pltpu.SemaphoreType.DMA,
            pltpu.SemaphoreType.DMA,
            pltpu.SemaphoreType.DMA,
        ],
    )(_edge_body)
    return k(feat, e_src, e_tar, src_p, tar_p)


def _final_body(part_ref, s_ref, feat_ref, epre_ref, out_ref):
    ep = epre_ref[...]
    z = ep[:, 0] + ep[:, 1]
    eself = jnp.exp(jnp.maximum(z, NEG * z))
    feat = feat_ref[...]
    num = part_ref[0] + part_ref[1] + eself[:, None] * feat
    den = jnp.sum(s_ref[...], axis=1) + eself
    out_ref[...] = num / den[:, None]


def _finalize(part, s_t, feat, epre):
    blk = 1000
    return pl.pallas_call(
        _final_body,
        grid=(N // blk,),
        in_specs=[
            pl.BlockSpec((2, blk, D), lambda i: (0, i, 0)),
            pl.BlockSpec((blk, NWORK), lambda i: (i, 0)),
            pl.BlockSpec((blk, D), lambda i: (i, 0)),
            pl.BlockSpec((blk, 2), lambda i: (i, 0)),
        ],
        out_specs=pl.BlockSpec((blk, D), lambda i: (i, 0)),
        out_shape=jax.ShapeDtypeStruct((N, D), jnp.float32),
    )(part, s_t, feat, epre)


def kernel(x, edge_index, W, b, att):
    feat, epre = _project(x, W, b.reshape(1, D), att)

    dummy = N + (jnp.arange(EPAD - E, dtype=jnp.int32) % (NPAD - N))
    ei_p = jnp.concatenate(
        [edge_index, jnp.broadcast_to(dummy, (2, EPAD - E))], axis=1)
    src_p = ei_p[1].reshape(NWORK, CHUNKS, B)
    tar_p = ei_p[0].reshape(NWORK, CHUNKS, B)

    e_src = epre[:, 0]
    e_tar = epre[:, 1]

    part, s_part = _edge_aggregate(feat, e_src, e_tar, src_p, tar_p)
    s_t = s_part.reshape(NWORK, NPAD).T
    return _finalize(part, s_t, feat, epre)


# speedup vs baseline: 1.0038x; 1.0038x over previous
import functools

import jax
import jax.numpy as jnp
from jax import lax
from jax.experimental import pallas as pl
from jax.experimental.pallas import tpu as pltpu
from jax.experimental.pallas import tpu_sc as plsc

N = 10000
NPAD = 10112
E = 320000
EPAD = 331776
D = 128
NEG = 0.2
NWORK = 32
CHUNKS = 216
B = 48
SCH = 8
NSUP = CHUNKS // SCH
PAIRS = CHUNKS // 2
PPS = SCH // 2


def _proj_body(x_ref, w_ref, b_ref, att_ref, feat_ref, epre_ref):
    feat = jnp.dot(x_ref[...], w_ref[...], preferred_element_type=jnp.float32)
    feat = feat + b_ref[...]
    feat_ref[pl.ds(0, N), :] = feat
    feat_ref[pl.ds(N, NPAD - N), :] = jnp.broadcast_to(b_ref[...],
                                                       (NPAD - N, D))
    epre = jnp.dot(feat, att_ref[...], preferred_element_type=jnp.float32)
    epre_ref[pl.ds(0, N), :] = epre
    epre_ref[pl.ds(N, NPAD - N), :] = jnp.zeros((NPAD - N, 2), jnp.float32)


def _project(x, W, b2, att):
    return pl.pallas_call(
        _proj_body,
        out_shape=(
            jax.ShapeDtypeStruct((NPAD, D), jnp.float32),
            jax.ShapeDtypeStruct((NPAD, 2), jnp.float32),
        ),
    )(x, W, b2, att)


def _edge_body(feat_hbm, esrc_hbm, etar_hbm, src_hbm, tar_hbm,
               out_ref, outs_ref,
               acc_s, esrc_v, etar_v, srcv, tarv, rows0, rows1, wbuf, s_v,
               gsem0, gsem1, ssem0, ssem1, isem):
    cid = lax.axis_index("c")
    sid = lax.axis_index("s")
    wid = cid * 16 + sid

    pltpu.sync_copy(esrc_hbm, esrc_v)
    pltpu.sync_copy(etar_hbm, etar_v)

    zero16 = jnp.zeros((16,), jnp.float32)
    lane = lax.broadcasted_iota(jnp.int32, (16,), 0)

    def _zs(j, _):
        s_v[pl.ds(j * 16, 16)] = zero16
        return 0

    lax.fori_loop(0, NPAD // 16, _zs, 0)

    def _zrow(j, _):
        for c in range(D // 16):
            rows0[j, pl.ds(c * 16, 16)] = zero16
        return 0

    lax.fori_loop(0, B, _zrow, 0)
    rows_per_tile = NPAD // 16
    base = sid * rows_per_tile
    for k in range(rows_per_tile // B):
        pltpu.sync_copy(rows0, acc_s.at[pl.ds(base + k * B, B)])
    rem = rows_per_tile % B
    if rem:
        pltpu.sync_copy(rows0.at[pl.ds(0, rem)],
                        acc_s.at[pl.ds(base + (rows_per_tile // B) * B, rem)])
    plsc.subcore_barrier()

    def _refill_issue(sup, half):
        pltpu.async_copy(src_hbm.at[wid, pl.ds(sup * SCH, SCH)],
                         srcv.at[half], isem)
        pltpu.async_copy(tar_hbm.at[wid, pl.ds(sup * SCH, SCH)],
                         tarv.at[half], isem)

    def _refill_wait(half):
        pltpu.make_async_copy(src_hbm.at[wid, pl.ds(0, SCH)],
                              srcv.at[half], isem).wait()
        pltpu.make_async_copy(tar_hbm.at[wid, pl.ds(0, SCH)],
                              tarv.at[half], isem).wait()

    def _weights(h, gg):
        for i in range(B // 16):
            s16 = srcv[h, gg, pl.ds(i * 16, 16)]
            t16 = tarv[h, gg, pl.ds(i * 16, 16)]
            z = (plsc.load_gather(esrc_v, [s16])
                 + plsc.load_gather(etar_v, [t16]))
            w16 = jnp.exp(jnp.maximum(z, NEG * z))
            wbuf[pl.ds(i * 16, 16)] = w16
            plsc.addupdate_scatter(s_v, [t16], w16)

    def _scale(rows):
        def _edge(j, _):
            wsp = plsc.load_gather(wbuf, [lane * 0 + j])
            for c in range(D // 16):
                rows[j, pl.ds(c * 16, 16)] = rows[j, pl.ds(c * 16, 16)] * wsp
            return 0

        lax.fori_loop(0, B, _edge, 0, unroll=8)

    pltpu.sync_copy(src_hbm.at[wid, pl.ds(0, SCH)], srcv.at[0])
    pltpu.sync_copy(tar_hbm.at[wid, pl.ds(0, SCH)], tarv.at[0])
    pltpu.async_copy(feat_hbm.at[srcv.at[0, 0]], rows0, gsem0)
    pltpu.async_copy(feat_hbm.at[srcv.at[0, 1]], rows1, gsem1)
    _refill_issue(1, 1)

    def _pair(p, _):
        sc = p // PPS
        pin = p % PPS
        h = sc & 1
        ga = (2 * p) % SCH
        gb = ga + 1

        @pl.when(jnp.logical_and(jnp.logical_and(pin == 0, p > 0),
                                 sc + 1 < NSUP))
        def _():
            _refill_issue(sc + 1, 1 - h)

        _weights(h, ga)
        pltpu.make_async_copy(feat_hbm.at[srcv.at[h, ga]], rows0, gsem0).wait()
        _scale(rows0)
        d_sa = pltpu.async_copy(rows0, acc_s.at[tarv.at[h, ga]], ssem0,
                                add=True)

        _weights(h, gb)
        pltpu.make_async_copy(feat_hbm.at[srcv.at[h, gb]], rows1, gsem1).wait()
        _scale(rows1)
        d_sb = pltpu.async_copy(rows1, acc_s.at[tarv.at[h, gb]], ssem1,
                                add=True)

        d_sa.wait()

        @pl.when(jnp.logical_and(pin == PPS - 1, p + 1 < PAIRS))
        def _():
            _refill_wait(1 - h)

        @pl.when(p + 1 < PAIRS)
        def _():
            nh = ((p + 1) // PPS) & 1
            nga = (2 * p + 2) % SCH
            pltpu.async_copy(feat_hbm.at[srcv.at[nh, nga]], rows0, gsem0)

        d_sb.wait()

        @pl.when(p + 1 < PAIRS)
        def _():
            nh = ((p + 1) // PPS) & 1
            ngb = (2 * p + 3) % SCH
            pltpu.async_copy(feat_hbm.at[srcv.at[nh, ngb]], rows1, gsem1)
        return 0

    lax.fori_loop(0, PAIRS, _pair, 0)
    plsc.subcore_barrier()

    for k in range(rows_per_tile // B):
        off = base + k * B
        pltpu.sync_copy(acc_s.at[pl.ds(off, B)], out_ref.at[cid, pl.ds(off, B)])
    if rem:
        off = base + (rows_per_tile // B) * B
        pltpu.sync_copy(acc_s.at[pl.ds(off, rem)],
                        out_ref.at[cid, pl.ds(off, rem)])
    pltpu.sync_copy(s_v, outs_ref.at[cid, sid])


def _edge_aggregate(feat, e_src, e_tar, src_p, tar_p):
    mesh = plsc.VectorSubcoreMesh(core_axis_name="c", subcore_axis_name="s")
    k = functools.partial(
        pl.kernel,
        out_type=(
            jax.ShapeDtypeStruct((2, NPAD, D), jnp.float32),
            jax.ShapeDtypeStruct((2, 16, NPAD), jnp.float32),
        ),
        mesh=mesh,
        compiler_params=pltpu.CompilerParams(needs_layout_passes=False),
        scratch_types=[
            pltpu.VMEM_SHARED((NPAD, D), jnp.float32),
            pltpu.VMEM((NPAD,), jnp.float32),
            pltpu.VMEM((NPAD,), jnp.float32),
            pltpu.VMEM((2, SCH, B), jnp.int32),
            pltpu.VMEM((2, SCH, B), jnp.int32),
            pltpu.VMEM((B, D), jnp.float32),
            pltpu.VMEM((B, D), jnp.float32),
            pltpu.VMEM((B,), jnp.float32),
            pltpu.VMEM((NPAD,), jnp.float32),
            pltpu.SemaphoreType.DMA,
            pltpu.SemaphoreType.DMA,
            pltpu.SemaphoreType.DMA,
            pltpu.SemaphoreType.DMA,
            pltpu.SemaphoreType.DMA,
        ],
    )(_edge_body)
    return k(feat, e_src, e_tar, src_p, tar_p)


def _final_body(part_ref, s_ref, feat_ref, epre_ref, out_ref):
    ep = epre_ref[...]
    z = ep[:, 0] + ep[:, 1]
    eself = jnp.exp(jnp.maximum(z, NEG * z))
    feat = feat_ref[...]
    num = part_ref[0] + part_ref[1] + eself[:, None] * feat
    den = jnp.sum(s_ref[...], axis=1) + eself
    out_ref[...] = num / den[:, None]


def _finalize(part, s_t, feat, epre):
    blk = 1000
    return pl.pallas_call(
        _final_body,
        grid=(N // blk,),
        in_specs=[
            pl.BlockSpec((2, blk, D), lambda i: (0, i, 0)),
            pl.BlockSpec((blk, NWORK), lambda i: (i, 0)),
            pl.BlockSpec((blk, D), lambda i: (i, 0)),
            pl.BlockSpec((blk, 2), lambda i: (i, 0)),
        ],
        out_specs=pl.BlockSpec((blk, D), lambda i: (i, 0)),
        out_shape=jax.ShapeDtypeStruct((N, D), jnp.float32),
    )(part, s_t, feat, epre)


def kernel(x, edge_index, W, b, att):
    feat, epre = _project(x, W, b.reshape(1, D), att)

    dummy = N + (jnp.arange(EPAD - E, dtype=jnp.int32) % (NPAD - N))
    ei_p = jnp.concatenate(
        [edge_index, jnp.broadcast_to(dummy, (2, EPAD - E))], axis=1)
    src_p = ei_p[1].reshape(NWORK, CHUNKS, B)
    tar_p = ei_p[0].reshape(NWORK, CHUNKS, B)

    e_src = epre[:, 0]
    e_tar = epre[:, 1]

    part, s_part = _edge_aggregate(feat, e_src, e_tar, src_p, tar_p)
    s_t = s_part.reshape(NWORK, NPAD).T
    return _finalize(part, s_t, feat, epre)


# speedup vs baseline: 1.0371x; 1.0331x over previous
import functools

import jax
import jax.numpy as jnp
from jax import lax
from jax.experimental import pallas as pl
from jax.experimental.pallas import tpu as pltpu
from jax.experimental.pallas import tpu_sc as plsc

N = 10000
NPAD = 10112
E = 320000
EPAD = 331776
D = 128
NEG = 0.2
NWORK = 32
CHUNKS = 216
B = 48
SCH = 8
NSUP = CHUNKS // SCH
PAIRS = CHUNKS // 2
PPS = SCH // 2


def _proj_body(x_ref, w_ref, b_ref, att_ref, feat_ref, epre_ref):
    feat = jnp.dot(x_ref[...], w_ref[...], preferred_element_type=jnp.float32)
    feat = feat + b_ref[...]
    feat_ref[pl.ds(0, N), :] = feat
    feat_ref[pl.ds(N, NPAD - N), :] = jnp.broadcast_to(b_ref[...],
                                                       (NPAD - N, D))
    epre = jnp.dot(feat, att_ref[...], preferred_element_type=jnp.float32)
    epre_ref[pl.ds(0, N), :] = epre
    epre_ref[pl.ds(N, NPAD - N), :] = jnp.zeros((NPAD - N, 2), jnp.float32)


def _project(x, W, b2, att):
    return pl.pallas_call(
        _proj_body,
        out_shape=(
            jax.ShapeDtypeStruct((NPAD, D), jnp.float32),
            jax.ShapeDtypeStruct((NPAD, 2), jnp.float32),
        ),
    )(x, W, b2, att)


def _edge_body(feat_hbm, esrc_hbm, etar_hbm, src_hbm, tar_hbm,
               out_ref, outs_ref,
               acc_s, esrc_v, etar_v, srcv, tarv, rows0, rows1, wbuf, s_v,
               gsem0, gsem1, ssem0, ssem1, isem):
    cid = lax.axis_index("c")
    sid = lax.axis_index("s")
    wid = cid * 16 + sid

    d_es = pltpu.async_copy(esrc_hbm, esrc_v, isem)
    d_et = pltpu.async_copy(etar_hbm, etar_v, isem)

    zero16 = jnp.zeros((16,), jnp.float32)
    lane = lax.broadcasted_iota(jnp.int32, (16,), 0)

    def _zs(j, _):
        s_v[pl.ds(j * 16, 16)] = zero16
        return 0

    lax.fori_loop(0, NPAD // 16, _zs, 0)

    def _zrow(j, _):
        for c in range(D // 16):
            rows0[j, pl.ds(c * 16, 16)] = zero16
        return 0

    lax.fori_loop(0, B, _zrow, 0)
    rows_per_tile = NPAD // 16
    base = sid * rows_per_tile
    zds = []
    for k in range(rows_per_tile // B):
        zds.append(pltpu.async_copy(rows0, acc_s.at[pl.ds(base + k * B, B)],
                                    gsem0))
    rem = rows_per_tile % B
    if rem:
        zds.append(pltpu.async_copy(
            rows0.at[pl.ds(0, rem)],
            acc_s.at[pl.ds(base + (rows_per_tile // B) * B, rem)], gsem0))
    d_es.wait()
    d_et.wait()
    for d in zds:
        d.wait()
    plsc.subcore_barrier()

    def _refill_issue(sup, half):
        pltpu.async_copy(src_hbm.at[wid, pl.ds(sup * SCH, SCH)],
                         srcv.at[half], isem)
        pltpu.async_copy(tar_hbm.at[wid, pl.ds(sup * SCH, SCH)],
                         tarv.at[half], isem)

    def _refill_wait(half):
        pltpu.make_async_copy(src_hbm.at[wid, pl.ds(0, SCH)],
                              srcv.at[half], isem).wait()
        pltpu.make_async_copy(tar_hbm.at[wid, pl.ds(0, SCH)],
                              tarv.at[half], isem).wait()

    def _weights(h, gg):
        for i in range(B // 16):
            s16 = srcv[h, gg, pl.ds(i * 16, 16)]
            t16 = tarv[h, gg, pl.ds(i * 16, 16)]
            z = (plsc.load_gather(esrc_v, [s16])
                 + plsc.load_gather(etar_v, [t16]))
            w16 = jnp.exp(jnp.maximum(z, NEG * z))
            wbuf[pl.ds(i * 16, 16)] = w16
            plsc.addupdate_scatter(s_v, [t16], w16)

    def _scale(rows):
        def _edge(j, _):
            wsp = plsc.load_gather(wbuf, [lane * 0 + j])
            for c in range(D // 16):
                rows[j, pl.ds(c * 16, 16)] = rows[j, pl.ds(c * 16, 16)] * wsp
            return 0

        lax.fori_loop(0, B, _edge, 0, unroll=8)

    pltpu.sync_copy(src_hbm.at[wid, pl.ds(0, SCH)], srcv.at[0])
    pltpu.sync_copy(tar_hbm.at[wid, pl.ds(0, SCH)], tarv.at[0])
    pltpu.async_copy(feat_hbm.at[srcv.at[0, 0]], rows0, gsem0)
    pltpu.async_copy(feat_hbm.at[srcv.at[0, 1]], rows1, gsem1)
    _refill_issue(1, 1)

    def _pair(p, _):
        sc = p // PPS
        pin = p % PPS
        h = sc & 1
        ga = (2 * p) % SCH
        gb = ga + 1

        @pl.when(jnp.logical_and(jnp.logical_and(pin == 0, p > 0),
                                 sc + 1 < NSUP))
        def _():
            _refill_issue(sc + 1, 1 - h)

        _weights(h, ga)
        pltpu.make_async_copy(feat_hbm.at[srcv.at[h, ga]], rows0, gsem0).wait()
        _scale(rows0)
        d_sa = pltpu.async_copy(rows0, acc_s.at[tarv.at[h, ga]], ssem0,
                                add=True)

        _weights(h, gb)
        pltpu.make_async_copy(feat_hbm.at[srcv.at[h, gb]], rows1, gsem1).wait()
        _scale(rows1)
        d_sb = pltpu.async_copy(rows1, acc_s.at[tarv.at[h, gb]], ssem1,
                                add=True)

        d_sa.wait()

        @pl.when(jnp.logical_and(pin == PPS - 1, p + 1 < PAIRS))
        def _():
            _refill_wait(1 - h)

        @pl.when(p + 1 < PAIRS)
        def _():
            nh = ((p + 1) // PPS) & 1
            nga = (2 * p + 2) % SCH
            pltpu.async_copy(feat_hbm.at[srcv.at[nh, nga]], rows0, gsem0)

        d_sb.wait()

        @pl.when(p + 1 < PAIRS)
        def _():
            nh = ((p + 1) // PPS) & 1
            ngb = (2 * p + 3) % SCH
            pltpu.async_copy(feat_hbm.at[srcv.at[nh, ngb]], rows1, gsem1)
        return 0

    lax.fori_loop(0, PAIRS, _pair, 0)
    plsc.subcore_barrier()

    wds = [pltpu.async_copy(s_v, outs_ref.at[cid, sid], gsem1)]
    for k in range(rows_per_tile // B):
        off = base + k * B
        wds.append(pltpu.async_copy(acc_s.at[pl.ds(off, B)],
                                    out_ref.at[cid, pl.ds(off, B)], gsem0))
    if rem:
        off = base + (rows_per_tile // B) * B
        wds.append(pltpu.async_copy(acc_s.at[pl.ds(off, rem)],
                                    out_ref.at[cid, pl.ds(off, rem)], gsem0))
    for d in wds:
        d.wait()


def _edge_aggregate(feat, e_src, e_tar, src_p, tar_p):
    mesh = plsc.VectorSubcoreMesh(core_axis_name="c", subcore_axis_name="s")
    k = functools.partial(
        pl.kernel,
        out_type=(
            jax.ShapeDtypeStruct((2, NPAD, D), jnp.float32),
            jax.ShapeDtypeStruct((2, 16, NPAD), jnp.float32),
        ),
        mesh=mesh,
        compiler_params=pltpu.CompilerParams(needs_layout_passes=False),
        scratch_types=[
            pltpu.VMEM_SHARED((NPAD, D), jnp.float32),
            pltpu.VMEM((NPAD,), jnp.float32),
            pltpu.VMEM((NPAD,), jnp.float32),
            pltpu.VMEM((2, SCH, B), jnp.int32),
            pltpu.VMEM((2, SCH, B), jnp.int32),
            pltpu.VMEM((B, D), jnp.float32),
            pltpu.VMEM((B, D), jnp.float32),
            pltpu.VMEM((B,), jnp.float32),
            pltpu.VMEM((NPAD,), jnp.float32),
            pltpu.SemaphoreType.DMA,
            pltpu.SemaphoreType.DMA,
            pltpu.SemaphoreType.DMA,
            pltpu.SemaphoreType.DMA,
            pltpu.SemaphoreType.DMA,
        ],
    )(_edge_body)
    return k(feat, e_src, e_tar, src_p, tar_p)


def _final_body(part_ref, s_ref, feat_ref, epre_ref, out_ref):
    ep = epre_ref[...]
    z = ep[:, 0] + ep[:, 1]
    eself = jnp.exp(jnp.maximum(z, NEG * z))
    feat = feat_ref[...]
    num = part_ref[0] + part_ref[1] + eself[:, None] * feat
    den = jnp.sum(s_ref[...], axis=1) + eself
    out_ref[...] = num / den[:, None]


def _finalize(part, s_t, feat, epre):
    blk = 1000
    return pl.pallas_call(
        _final_body,
        grid=(N // blk,),
        in_specs=[
            pl.BlockSpec((2, blk, D), lambda i: (0, i, 0)),
            pl.BlockSpec((blk, NWORK), lambda i: (i, 0)),
            pl.BlockSpec((blk, D), lambda i: (i, 0)),
            pl.BlockSpec((blk, 2), lambda i: (i, 0)),
        ],
        out_specs=pl.BlockSpec((blk, D), lambda i: (i, 0)),
        out_shape=jax.ShapeDtypeStruct((N, D), jnp.float32),
    )(part, s_t, feat, epre)


def kernel(x, edge_index, W, b, att):
    feat, epre = _project(x, W, b.reshape(1, D), att)

    dummy = N + (jnp.arange(EPAD - E, dtype=jnp.int32) % (NPAD - N))
    ei_p = jnp.concatenate(
        [edge_index, jnp.broadcast_to(dummy, (2, EPAD - E))], axis=1)
    src_p = ei_p[1].reshape(NWORK, CHUNKS, B)
    tar_p = ei_p[0].reshape(NWORK, CHUNKS, B)

    e_src = epre[:, 0]
    e_tar = epre[:, 1]

    part, s_part = _edge_aggregate(feat, e_src, e_tar, src_p, tar_p)
    s_t = s_part.reshape(NWORK, NPAD).T
    return _finalize(part, s_t, feat, epre)
